# baseline (speedup 1.0000x reference)
import functools

import jax
import jax.numpy as jnp
from jax.experimental import pallas as pl
from jax.experimental.pallas import tpu as pltpu

_BT = 256
_GG = 8
_GC = 8


def _gather_body(n_in, src_ref, *refs):
    o_ref = refs[n_in]
    for g in range(n_in):
        o_ref[g, :] = refs[g][0, 0, :].astype(o_ref.dtype)


def _g1_body(bval_ref, brow_ref, bexp_ref, x_ref, wg_ref, wu_ref, h_ref):
    b = pl.program_id(1)

    @pl.when(bval_ref[b] == 1)
    def _():
        x = x_ref[...].astype(jnp.float32)
        g = jax.lax.dot_general(
            x, wg_ref[0, 0], (((1,), (1,)), ((), ())),
            preferred_element_type=jnp.float32)
        u = jax.lax.dot_general(
            x, wu_ref[0, 0], (((1,), (1,)), ((), ())),
            preferred_element_type=jnp.float32)
        h_ref[...] = ((g * jax.nn.sigmoid(g)) * u).astype(h_ref.dtype)


def _g2_body(bval_ref, brow_ref, bexp_ref, h_ref, w2_ref, o_ref):
    b = pl.program_id(0)

    @pl.when(bval_ref[b] == 1)
    def _():
        h = h_ref[...].astype(jnp.float32)
        o_ref[...] = jax.lax.dot_general(
            h, w2_ref[0], (((1,), (1,)), ((), ())),
            preferred_element_type=jnp.float32)


def _combine_body(n_in, pos_ref, w_ref, *refs):
    o_ref = refs[n_in]
    i = pl.program_id(0)
    k = n_in // _GC
    for g in range(_GC):
        j = (i * _GC + g) * k
        acc = w_ref[j] * refs[g * k][0, 0, :]
        for kk in range(1, k):
            acc = acc + w_ref[j + kk] * refs[g * k + kk][0, 0, :]
        o_ref[g, :] = acc


def kernel(hidden_states, topk_weights, topk_ids, w13_weight, w2_weight):
    T, H = hidden_states.shape
    _, K = topk_ids.shape
    E = w13_weight.shape[0]
    I = w2_weight.shape[2]
    N = T * K
    BT = _BT
    P = N + E * BT
    nb = P // BT

    ids = topk_ids.reshape(-1).astype(jnp.int32)
    sort_idx = jnp.argsort(ids).astype(jnp.int32)
    e_sorted = ids[sort_idx]
    counts = jnp.bincount(ids, length=E).astype(jnp.int32)
    off = jnp.concatenate([jnp.zeros((1,), jnp.int32),
                           jnp.cumsum(counts)[:-1].astype(jnp.int32)])
    pcnt = ((counts + BT - 1) // BT) * BT
    poff_full = jnp.concatenate([jnp.zeros((1,), jnp.int32),
                                 jnp.cumsum(pcnt).astype(jnp.int32)])
    poff = poff_full[:-1]
    ptotal = poff_full[-1]

    ppos = poff[e_sorted] + (jnp.arange(N, dtype=jnp.int32) - off[e_sorted])
    src = jnp.zeros((P,), jnp.int32).at[ppos].set(sort_idx // K)
    pos = jnp.zeros((N,), jnp.int32).at[sort_idx].set(ppos)

    b_idx = jnp.arange(nb, dtype=jnp.int32)
    nvalid = ptotal // BT
    valid = b_idx < nvalid
    bexp_raw = jnp.clip(
        jnp.searchsorted(poff_full, b_idx * BT, side='right').astype(jnp.int32)
        - 1, 0, E - 1)
    last = nvalid - 1
    brow = jnp.where(valid, b_idx, last).astype(jnp.int32)
    bexp = jnp.where(valid, bexp_raw, bexp_raw[last]).astype(jnp.int32)
    bval = valid.astype(jnp.int32)

    GG = _GG
    gsteps = P // GG
    hs3 = hidden_states.reshape(T, 1, H)
    gather = pl.pallas_call(
        functools.partial(_gather_body, GG),
        grid_spec=pltpu.PrefetchScalarGridSpec(
            num_scalar_prefetch=1,
            grid=(gsteps,),
            in_specs=[
                pl.BlockSpec((1, 1, H),
                             (lambda i, s, g=g: (s[i * GG + g], 0, 0)))
                for g in range(GG)
            ],
            out_specs=pl.BlockSpec((GG, H), lambda i, s: (i, 0)),
        ),
        out_shape=jax.ShapeDtypeStruct((P, H), jnp.bfloat16),
    )
    xd = gather(src, *([hs3] * GG))

    BI = I // 2
    nj = I // BI
    w13r = w13_weight.reshape(E, 2 * nj, BI, H)

    g1 = pl.pallas_call(
        _g1_body,
        grid_spec=pltpu.PrefetchScalarGridSpec(
            num_scalar_prefetch=3,
            grid=(nj, nb),
            in_specs=[
                pl.BlockSpec((BT, H),
                             lambda j, b, bv, br, be: (br[b], 0)),
                pl.BlockSpec((1, 1, BI, H),
                             lambda j, b, bv, br, be: (be[b], j, 0, 0)),
                pl.BlockSpec((1, 1, BI, H),
                             lambda j, b, bv, br, be: (be[b], nj + j, 0, 0)),
            ],
            out_specs=pl.BlockSpec((BT, BI),
                                   lambda j, b, bv, br, be: (br[b], j)),
        ),
        out_shape=jax.ShapeDtypeStruct((P, I), jnp.bfloat16),
        compiler_params=pltpu.CompilerParams(
            dimension_semantics=("arbitrary", "arbitrary"),
        ),
    )
    hmat = g1(bval, brow, bexp, xd, w13r, w13r)

    g2 = pl.pallas_call(
        _g2_body,
        grid_spec=pltpu.PrefetchScalarGridSpec(
            num_scalar_prefetch=3,
            grid=(nb,),
            in_specs=[
                pl.BlockSpec((BT, I),
                             lambda b, bv, br, be: (br[b], 0)),
                pl.BlockSpec((1, H, I),
                             lambda b, bv, br, be: (be[b], 0, 0)),
            ],
            out_specs=pl.BlockSpec((BT, H),
                                   lambda b, bv, br, be: (br[b], 0)),
        ),
        out_shape=jax.ShapeDtypeStruct((P, H), jnp.float32),
        compiler_params=pltpu.CompilerParams(
            dimension_semantics=("arbitrary",),
        ),
    )
    out_d = g2(bval, brow, bexp, hmat, w2_weight)

    GC = _GC
    csteps = T // GC
    n_in = GC * K
    out_d3 = out_d.reshape(P, 1, H)
    wflat = topk_weights.reshape(-1)
    combine = pl.pallas_call(
        functools.partial(_combine_body, n_in),
        grid_spec=pltpu.PrefetchScalarGridSpec(
            num_scalar_prefetch=2,
            grid=(csteps,),
            in_specs=[
                pl.BlockSpec((1, 1, H),
                             (lambda i, p, w, g=g, kk=kk:
                              (p[(i * GC + g) * K + kk], 0, 0)))
                for g in range(GC) for kk in range(K)
            ],
            out_specs=pl.BlockSpec((GC, H), lambda i, p, w: (i, 0)),
        ),
        out_shape=jax.ShapeDtypeStruct((T, H), jnp.float32),
    )
    return combine(pos, wflat, *([out_d3] * n_in))


# speedup vs baseline: 2.2433x; 2.2433x over previous
import functools

import jax
import jax.numpy as jnp
from jax import lax
from jax.experimental import pallas as pl
from jax.experimental.pallas import tpu as pltpu
from jax.experimental.pallas import tpu_sc as plsc

_BT = 256
_NW = 32


def _dispatch_body(K, chunk, pos_hbm, hid_hbm, xd_hbm,
                   ppos_v, rows_v, tok_v, sidx_v, sem_g, sem_s):
    NC = 2
    wid = lax.axis_index("s") * NC + lax.axis_index("c")
    nv = chunk // 16

    pltpu.sync_copy(pos_hbm.at[wid], ppos_v)

    kshift = K.bit_length() - 1
    lane_div = jnp.arange(16, dtype=jnp.int32) >> kshift
    for c in range(nv):
        tok_v[...] = lane_div + (wid * (chunk // K) + c * (16 // K))
        sidx_v[...] = ppos_v[c, :]
        buf = c % 2
        pltpu.async_copy(hid_hbm.at[tok_v], rows_v.at[buf], sem_g).wait()
        pltpu.async_copy(rows_v.at[buf], xd_hbm.at[sidx_v], sem_s).wait()


def _sc_combine_body(K, H, tchunk, pos_hbm, w_hbm, yd_hbm, out_hbm,
                     posc_v, w_v, rows_v, obuf_v, sidx_v, sem_g):
    NC = 2
    wid = lax.axis_index("s") * NC + lax.axis_index("c")
    nv = (tchunk * K) // 16
    tpc = 16 // K

    pltpu.sync_copy(pos_hbm.at[wid], posc_v)
    pltpu.sync_copy(w_hbm.at[wid], w_v)

    zf = jnp.zeros((16,), jnp.float32)
    for c in range(nv):
        sidx_v[...] = posc_v[c, :]
        pltpu.async_copy(yd_hbm.at[sidx_v], rows_v, sem_g).wait()
        wrow = w_v[c, :]
        wvecs = [zf + wrow[i] for i in range(16)]

        def col_body(l, _):
            for g in range(tpc):
                acc = rows_v[g * K, pl.ds(l * 16, 16)] * wvecs[g * K]
                for kk in range(1, K):
                    acc = acc + (rows_v[g * K + kk, pl.ds(l * 16, 16)]
                                 * wvecs[g * K + kk])
                obuf_v[g, pl.ds(l * 16, 16)] = acc
            return 0

        lax.fori_loop(0, H // 16, col_body, 0)
        pltpu.sync_copy(obuf_v, out_hbm.at[pl.ds(wid * tchunk + c * tpc,
                                                 tpc)])


def _g1_body(tab_ref, x_ref, wg_ref, wu_ref, h_ref):
    b = pl.program_id(1)

    @pl.when(tab_ref[0, b] == 1)
    def _():
        x = x_ref[...]
        g = jax.lax.dot_general(
            x, wg_ref[0, 0], (((1,), (1,)), ((), ())),
            preferred_element_type=jnp.float32)
        u = jax.lax.dot_general(
            x, wu_ref[0, 0], (((1,), (1,)), ((), ())),
            preferred_element_type=jnp.float32)
        h_ref[...] = ((g * jax.nn.sigmoid(g)) * u).astype(h_ref.dtype)


def _g2_body(tab_ref, h_ref, w2_ref, o_ref):
    b = pl.program_id(0)

    @pl.when(tab_ref[0, b] == 1)
    def _():
        h = h_ref[...].astype(jnp.float32)
        o_ref[...] = jax.lax.dot_general(
            h, w2_ref[0], (((1,), (1,)), ((), ())),
            preferred_element_type=jnp.float32)


def kernel(hidden_states, topk_weights, topk_ids, w13_weight, w2_weight):
    T, H = hidden_states.shape
    _, K = topk_ids.shape
    E = w13_weight.shape[0]
    I = w2_weight.shape[2]
    N = T * K
    BT = _BT
    P = N + E * BT
    nb = P // BT
    NW = _NW
    chunk = N // NW

    ids = topk_ids.reshape(-1).astype(jnp.int32)
    sort_idx = jnp.argsort(ids).astype(jnp.int32)
    e_sorted = ids[sort_idx]
    counts = jnp.bincount(ids, length=E).astype(jnp.int32)
    off = jnp.concatenate([jnp.zeros((1,), jnp.int32),
                           jnp.cumsum(counts)[:-1].astype(jnp.int32)])
    pcnt = ((counts + BT - 1) // BT) * BT
    poff_full = jnp.concatenate([jnp.zeros((1,), jnp.int32),
                                 jnp.cumsum(pcnt).astype(jnp.int32)])
    poff = poff_full[:-1]
    ptotal = poff_full[-1]

    ppos = poff[e_sorted] + (jnp.arange(N, dtype=jnp.int32) - off[e_sorted])
    pos = jnp.zeros((N,), jnp.int32).at[sort_idx].set(ppos)

    b_idx = jnp.arange(nb, dtype=jnp.int32)
    nvalid = ptotal // BT
    valid = b_idx < nvalid
    bexp_raw = jnp.clip(
        jnp.searchsorted(poff_full, b_idx * BT, side='right').astype(jnp.int32)
        - 1, 0, E - 1)
    last = nvalid - 1
    brow = jnp.where(valid, b_idx, last).astype(jnp.int32)
    bexp = jnp.where(valid, bexp_raw, bexp_raw[last]).astype(jnp.int32)
    tab = jnp.stack([valid.astype(jnp.int32), brow, bexp])

    pos3 = pos.reshape(NW, chunk // 16, 16)
    mesh = plsc.VectorSubcoreMesh(core_axis_name="c", subcore_axis_name="s")
    dispatch = pl.kernel(
        functools.partial(_dispatch_body, K, chunk),
        mesh=mesh,
        out_type=[jax.ShapeDtypeStruct((P, H), jnp.float32)],
        scratch_types=[
            pltpu.VMEM((chunk // 16, 16), jnp.int32),
            pltpu.VMEM((2, 16, H), jnp.float32),
            pltpu.VMEM((16,), jnp.int32),
            pltpu.VMEM((16,), jnp.int32),
            pltpu.SemaphoreType.DMA,
            pltpu.SemaphoreType.DMA,
        ],
    )
    (xd,) = dispatch(pos3, hidden_states)

    BI = I // 2
    nj = I // BI
    w13r = w13_weight.reshape(E, 2 * nj, BI, H)

    g1 = pl.pallas_call(
        _g1_body,
        grid_spec=pltpu.PrefetchScalarGridSpec(
            num_scalar_prefetch=1,
            grid=(nj, nb),
            in_specs=[
                pl.BlockSpec((BT, H),
                             lambda j, b, tb: (tb[1, b], 0)),
                pl.BlockSpec((1, 1, BI, H),
                             lambda j, b, tb: (tb[2, b], j, 0, 0)),
                pl.BlockSpec((1, 1, BI, H),
                             lambda j, b, tb: (tb[2, b], nj + j, 0, 0)),
            ],
            out_specs=pl.BlockSpec((BT, BI),
                                   lambda j, b, tb: (tb[1, b], j)),
        ),
        out_shape=jax.ShapeDtypeStruct((P, I), jnp.bfloat16),
        compiler_params=pltpu.CompilerParams(
            dimension_semantics=("arbitrary", "arbitrary"),
        ),
    )
    hmat = g1(tab, xd, w13r, w13r)

    g2 = pl.pallas_call(
        _g2_body,
        grid_spec=pltpu.PrefetchScalarGridSpec(
            num_scalar_prefetch=1,
            grid=(nb,),
            in_specs=[
                pl.BlockSpec((BT, I),
                             lambda b, tb: (tb[1, b], 0)),
                pl.BlockSpec((1, H, I),
                             lambda b, tb: (tb[2, b], 0, 0)),
            ],
            out_specs=pl.BlockSpec((BT, H),
                                   lambda b, tb: (tb[1, b], 0)),
        ),
        out_shape=jax.ShapeDtypeStruct((P, H), jnp.float32),
        compiler_params=pltpu.CompilerParams(
            dimension_semantics=("arbitrary",),
        ),
    )
    out_d = g2(tab, hmat, w2_weight)

    tchunk = T // NW
    w3 = topk_weights.reshape(NW, (tchunk * K) // 16, 16)
    combine = pl.kernel(
        functools.partial(_sc_combine_body, K, H, tchunk),
        mesh=mesh,
        out_type=[jax.ShapeDtypeStruct((T, H), jnp.float32)],
        scratch_types=[
            pltpu.VMEM(((tchunk * K) // 16, 16), jnp.int32),
            pltpu.VMEM(((tchunk * K) // 16, 16), jnp.float32),
            pltpu.VMEM((16, H), jnp.float32),
            pltpu.VMEM((16 // K, H), jnp.float32),
            pltpu.VMEM((16,), jnp.int32),
            pltpu.SemaphoreType.DMA,
        ],
    )
    (out,) = combine(pos3, w3, out_d)
    return out


# speedup vs baseline: 2.4063x; 1.0727x over previous
import functools

import jax
import jax.numpy as jnp
from jax import lax
from jax.experimental import pallas as pl
from jax.experimental.pallas import tpu as pltpu
from jax.experimental.pallas import tpu_sc as plsc

_BT = 256
_NW = 32


def _dispatch_body(K, chunk, pos_hbm, hid_hbm, xd_hbm,
                   ppos_v, rows_v, tok_v, sidx_v, sem_g, sem_s):
    NC = 2
    wid = lax.axis_index("s") * NC + lax.axis_index("c")
    nv = chunk // 16

    pltpu.sync_copy(pos_hbm.at[wid], ppos_v)

    kshift = K.bit_length() - 1
    lane_div = jnp.arange(16, dtype=jnp.int32) >> kshift
    for c in range(nv):
        tok_v[...] = lane_div + (wid * (chunk // K) + c * (16 // K))
        sidx_v[...] = ppos_v[c, :]
        buf = c % 2
        pltpu.async_copy(hid_hbm.at[tok_v], rows_v.at[buf], sem_g).wait()
        pltpu.async_copy(rows_v.at[buf], xd_hbm.at[sidx_v], sem_s).wait()


def _sc_combine_body(K, H, tchunk, pos_hbm, w_hbm, yd_hbm, out_hbm,
                     posc_v, w_v, rows_v, obuf_v, sidx_v, sem_g):
    NC = 2
    wid = lax.axis_index("s") * NC + lax.axis_index("c")
    nv = (tchunk * K) // 16
    tpc = 16 // K

    pltpu.sync_copy(pos_hbm.at[wid], posc_v)
    pltpu.sync_copy(w_hbm.at[wid], w_v)

    zf = jnp.zeros((16,), jnp.float32)
    for c in range(nv):
        sidx_v[...] = posc_v[c, :]
        pltpu.async_copy(yd_hbm.at[sidx_v], rows_v, sem_g).wait()
        wrow = w_v[c, :]
        wvecs = [zf + wrow[i] for i in range(16)]

        def col_body(l, _):
            for g in range(tpc):
                acc = rows_v[g * K, pl.ds(l * 16, 16)] * wvecs[g * K]
                for kk in range(1, K):
                    acc = acc + (rows_v[g * K + kk, pl.ds(l * 16, 16)]
                                 * wvecs[g * K + kk])
                obuf_v[g, pl.ds(l * 16, 16)] = acc
            return 0

        lax.fori_loop(0, H // 16, col_body, 0)
        pltpu.sync_copy(obuf_v, out_hbm.at[pl.ds(wid * tchunk + c * tpc,
                                                 tpc)])


def _g1_body(tab_ref, x_ref, wg_ref, wu_ref, h_ref):
    b = pl.program_id(1)

    @pl.when(tab_ref[0, b] == 1)
    def _():
        x = x_ref[...]
        g = jax.lax.dot_general(
            x, wg_ref[0, 0], (((1,), (1,)), ((), ())),
            preferred_element_type=jnp.float32)
        u = jax.lax.dot_general(
            x, wu_ref[0, 0], (((1,), (1,)), ((), ())),
            preferred_element_type=jnp.float32)
        h_ref[...] = ((g * jax.nn.sigmoid(g)) * u).astype(h_ref.dtype)


def _g2_body(tab_ref, h_ref, w2_ref, o_ref):
    b = pl.program_id(0)

    @pl.when(tab_ref[0, b] == 1)
    def _():
        h = h_ref[...].astype(jnp.float32)
        o_ref[...] = jax.lax.dot_general(
            h, w2_ref[0], (((1,), (1,)), ((), ())),
            preferred_element_type=jnp.float32)


def kernel(hidden_states, topk_weights, topk_ids, w13_weight, w2_weight):
    T, H = hidden_states.shape
    _, K = topk_ids.shape
    E = w13_weight.shape[0]
    I = w2_weight.shape[2]
    N = T * K
    BT = _BT
    P = N + E * BT
    nb = P // BT
    NW = _NW
    chunk = N // NW

    ids = topk_ids.reshape(-1).astype(jnp.int32)
    oh = (ids[:, None] == jnp.arange(E, dtype=jnp.int32)[None, :]
          ).astype(jnp.int32)
    csum = jnp.cumsum(oh, axis=0)
    counts = csum[-1]
    pcnt = ((counts + BT - 1) // BT) * BT
    poff_full = jnp.concatenate([jnp.zeros((1,), jnp.int32),
                                 jnp.cumsum(pcnt).astype(jnp.int32)])
    poff = poff_full[:-1]
    ptotal = poff_full[-1]

    rank = jnp.take_along_axis(csum, ids[:, None], axis=1)[:, 0] - 1
    pos = (poff[ids] + rank).astype(jnp.int32)

    b_idx = jnp.arange(nb, dtype=jnp.int32)
    nvalid = ptotal // BT
    valid = b_idx < nvalid
    bexp_raw = jnp.clip(
        jnp.searchsorted(poff_full, b_idx * BT, side='right').astype(jnp.int32)
        - 1, 0, E - 1)
    last = nvalid - 1
    brow = jnp.where(valid, b_idx, last).astype(jnp.int32)
    bexp = jnp.where(valid, bexp_raw, bexp_raw[last]).astype(jnp.int32)
    tab = jnp.stack([valid.astype(jnp.int32), brow, bexp])

    pos3 = pos.reshape(NW, chunk // 16, 16)
    mesh = plsc.VectorSubcoreMesh(core_axis_name="c", subcore_axis_name="s")
    dispatch = pl.kernel(
        functools.partial(_dispatch_body, K, chunk),
        mesh=mesh,
        out_type=[jax.ShapeDtypeStruct((P, H), jnp.float32)],
        scratch_types=[
            pltpu.VMEM((chunk // 16, 16), jnp.int32),
            pltpu.VMEM((2, 16, H), jnp.float32),
            pltpu.VMEM((16,), jnp.int32),
            pltpu.VMEM((16,), jnp.int32),
            pltpu.SemaphoreType.DMA,
            pltpu.SemaphoreType.DMA,
        ],
    )
    (xd,) = dispatch(pos3, hidden_states)

    BI = I // 2
    nj = I // BI
    w13r = w13_weight.reshape(E, 2 * nj, BI, H)

    g1 = pl.pallas_call(
        _g1_body,
        grid_spec=pltpu.PrefetchScalarGridSpec(
            num_scalar_prefetch=1,
            grid=(nj, nb),
            in_specs=[
                pl.BlockSpec((BT, H),
                             lambda j, b, tb: (tb[1, b], 0)),
                pl.BlockSpec((1, 1, BI, H),
                             lambda j, b, tb: (tb[2, b], j, 0, 0)),
                pl.BlockSpec((1, 1, BI, H),
                             lambda j, b, tb: (tb[2, b], nj + j, 0, 0)),
            ],
            out_specs=pl.BlockSpec((BT, BI),
                                   lambda j, b, tb: (tb[1, b], j)),
        ),
        out_shape=jax.ShapeDtypeStruct((P, I), jnp.bfloat16),
        compiler_params=pltpu.CompilerParams(
            dimension_semantics=("arbitrary", "arbitrary"),
        ),
    )
    hmat = g1(tab, xd, w13r, w13r)

    g2 = pl.pallas_call(
        _g2_body,
        grid_spec=pltpu.PrefetchScalarGridSpec(
            num_scalar_prefetch=1,
            grid=(nb,),
            in_specs=[
                pl.BlockSpec((BT, I),
                             lambda b, tb: (tb[1, b], 0)),
                pl.BlockSpec((1, H, I),
                             lambda b, tb: (tb[2, b], 0, 0)),
            ],
            out_specs=pl.BlockSpec((BT, H),
                                   lambda b, tb: (tb[1, b], 0)),
        ),
        out_shape=jax.ShapeDtypeStruct((P, H), jnp.float32),
        compiler_params=pltpu.CompilerParams(
            dimension_semantics=("arbitrary",),
        ),
    )
    out_d = g2(tab, hmat, w2_weight)

    tchunk = T // NW
    w3 = topk_weights.reshape(NW, (tchunk * K) // 16, 16)
    combine = pl.kernel(
        functools.partial(_sc_combine_body, K, H, tchunk),
        mesh=mesh,
        out_type=[jax.ShapeDtypeStruct((T, H), jnp.float32)],
        scratch_types=[
            pltpu.VMEM(((tchunk * K) // 16, 16), jnp.int32),
            pltpu.VMEM(((tchunk * K) // 16, 16), jnp.float32),
            pltpu.VMEM((16, H), jnp.float32),
            pltpu.VMEM((16 // K, H), jnp.float32),
            pltpu.VMEM((16,), jnp.int32),
            pltpu.SemaphoreType.DMA,
        ],
    )
    (out,) = combine(pos3, w3, out_d)
    return out


# speedup vs baseline: 2.4265x; 1.0084x over previous
import functools

import jax
import jax.numpy as jnp
from jax import lax
from jax.experimental import pallas as pl
from jax.experimental.pallas import tpu as pltpu
from jax.experimental.pallas import tpu_sc as plsc

_BT = 256
_NW = 32


def _dispatch_body(K, chunk, pos_hbm, hid_hbm, xd_hbm,
                   ppos_v, rows_v, tok_v, sidx_v, sem_g, sem_s):
    NC = 2
    wid = lax.axis_index("s") * NC + lax.axis_index("c")
    nv = chunk // 16

    pltpu.sync_copy(pos_hbm.at[wid], ppos_v)

    kshift = K.bit_length() - 1
    lane_div = jnp.arange(16, dtype=jnp.int32) >> kshift

    def gather(c):
        buf = c % 2
        tok_v[buf, :] = lane_div + (wid * (chunk // K) + c * (16 // K))
        return pltpu.async_copy(hid_hbm.at[tok_v.at[buf]], rows_v.at[buf],
                                sem_g)

    def scatter(c):
        buf = c % 2
        sidx_v[buf, :] = ppos_v[c, :]
        return pltpu.async_copy(rows_v.at[buf], xd_hbm.at[sidx_v.at[buf]],
                                sem_s)

    g_h = [None] * nv
    s_h = [None] * nv
    g_h[0] = gather(0)
    for c in range(nv):
        g_h[c].wait()
        s_h[c] = scatter(c)
        if c + 1 < nv:
            if c >= 1:
                s_h[c - 1].wait()
            g_h[c + 1] = gather(c + 1)
    if nv >= 2:
        s_h[nv - 2].wait()
    s_h[nv - 1].wait()


def _sc_combine_body(K, H, tchunk, pos_hbm, w_hbm, yd_hbm, out_hbm,
                     posc_v, w_v, rows_v, obuf_v, sidx_v, sem_g):
    NC = 2
    wid = lax.axis_index("s") * NC + lax.axis_index("c")
    nv = (tchunk * K) // 16
    tpc = 16 // K

    pltpu.sync_copy(pos_hbm.at[wid], posc_v)
    pltpu.sync_copy(w_hbm.at[wid], w_v)

    zf = jnp.zeros((16,), jnp.float32)

    def gather(c):
        buf = c % 2
        sidx_v[buf, :] = posc_v[c, :]
        return pltpu.async_copy(yd_hbm.at[sidx_v.at[buf]], rows_v.at[buf],
                                sem_g)

    g_h = [None] * nv
    g_h[0] = gather(0)
    for c in range(nv):
        g_h[c].wait()
        if c + 1 < nv:
            g_h[c + 1] = gather(c + 1)
        buf = c % 2
        wrow = w_v[c, :]
        wvecs = [zf + wrow[i] for i in range(16)]

        def col_body(l, _, buf=buf, wvecs=wvecs):
            for g in range(tpc):
                acc = rows_v[buf, g * K, pl.ds(l * 16, 16)] * wvecs[g * K]
                for kk in range(1, K):
                    acc = acc + (rows_v[buf, g * K + kk, pl.ds(l * 16, 16)]
                                 * wvecs[g * K + kk])
                obuf_v[g, pl.ds(l * 16, 16)] = acc
            return 0

        lax.fori_loop(0, H // 16, col_body, 0)
        pltpu.sync_copy(obuf_v, out_hbm.at[pl.ds(wid * tchunk + c * tpc,
                                                 tpc)])


def _g1_body(tab_ref, x_ref, wg_ref, wu_ref, h_ref):
    b = pl.program_id(1)

    @pl.when(tab_ref[0, b] == 1)
    def _():
        x = x_ref[...]
        g = jax.lax.dot_general(
            x, wg_ref[0, 0], (((1,), (1,)), ((), ())),
            preferred_element_type=jnp.float32)
        u = jax.lax.dot_general(
            x, wu_ref[0, 0], (((1,), (1,)), ((), ())),
            preferred_element_type=jnp.float32)
        h_ref[...] = ((g * jax.nn.sigmoid(g)) * u).astype(h_ref.dtype)


def _g2_body(tab_ref, h_ref, w2_ref, o_ref):
    b = pl.program_id(0)

    @pl.when(tab_ref[0, b] == 1)
    def _():
        h = h_ref[...].astype(jnp.float32)
        o_ref[...] = jax.lax.dot_general(
            h, w2_ref[0], (((1,), (1,)), ((), ())),
            preferred_element_type=jnp.float32)


def kernel(hidden_states, topk_weights, topk_ids, w13_weight, w2_weight):
    T, H = hidden_states.shape
    _, K = topk_ids.shape
    E = w13_weight.shape[0]
    I = w2_weight.shape[2]
    N = T * K
    BT = _BT
    P = N + E * BT
    nb = P // BT
    NW = _NW
    chunk = N // NW

    ids = topk_ids.reshape(-1).astype(jnp.int32)
    oh = (ids[:, None] == jnp.arange(E, dtype=jnp.int32)[None, :]
          ).astype(jnp.int32)
    csum = jnp.cumsum(oh, axis=0)
    counts = csum[-1]
    pcnt = ((counts + BT - 1) // BT) * BT
    poff_full = jnp.concatenate([jnp.zeros((1,), jnp.int32),
                                 jnp.cumsum(pcnt).astype(jnp.int32)])
    poff = poff_full[:-1]
    ptotal = poff_full[-1]

    rank = jnp.take_along_axis(csum, ids[:, None], axis=1)[:, 0] - 1
    pos = (poff[ids] + rank).astype(jnp.int32)

    b_idx = jnp.arange(nb, dtype=jnp.int32)
    nvalid = ptotal // BT
    valid = b_idx < nvalid
    bexp_raw = jnp.clip(
        jnp.searchsorted(poff_full, b_idx * BT, side='right').astype(jnp.int32)
        - 1, 0, E - 1)
    last = nvalid - 1
    brow = jnp.where(valid, b_idx, last).astype(jnp.int32)
    bexp = jnp.where(valid, bexp_raw, bexp_raw[last]).astype(jnp.int32)
    tab = jnp.stack([valid.astype(jnp.int32), brow, bexp])

    pos3 = pos.reshape(NW, chunk // 16, 16)
    mesh = plsc.VectorSubcoreMesh(core_axis_name="c", subcore_axis_name="s")
    dispatch = pl.kernel(
        functools.partial(_dispatch_body, K, chunk),
        mesh=mesh,
        out_type=[jax.ShapeDtypeStruct((P, H), jnp.float32)],
        scratch_types=[
            pltpu.VMEM((chunk // 16, 16), jnp.int32),
            pltpu.VMEM((2, 16, H), jnp.float32),
            pltpu.VMEM((2, 16), jnp.int32),
            pltpu.VMEM((2, 16), jnp.int32),
            pltpu.SemaphoreType.DMA,
            pltpu.SemaphoreType.DMA,
        ],
    )
    (xd,) = dispatch(pos3, hidden_states)

    BI = I // 2
    nj = I // BI
    w13r = w13_weight.reshape(E, 2 * nj, BI, H)

    g1 = pl.pallas_call(
        _g1_body,
        grid_spec=pltpu.PrefetchScalarGridSpec(
            num_scalar_prefetch=1,
            grid=(nj, nb),
            in_specs=[
                pl.BlockSpec((BT, H),
                             lambda j, b, tb: (tb[1, b], 0)),
                pl.BlockSpec((1, 1, BI, H),
                             lambda j, b, tb: (tb[2, b], j, 0, 0)),
                pl.BlockSpec((1, 1, BI, H),
                             lambda j, b, tb: (tb[2, b], nj + j, 0, 0)),
            ],
            out_specs=pl.BlockSpec((BT, BI),
                                   lambda j, b, tb: (tb[1, b], j)),
        ),
        out_shape=jax.ShapeDtypeStruct((P, I), jnp.bfloat16),
        compiler_params=pltpu.CompilerParams(
            dimension_semantics=("arbitrary", "arbitrary"),
        ),
    )
    hmat = g1(tab, xd, w13r, w13r)

    g2 = pl.pallas_call(
        _g2_body,
        grid_spec=pltpu.PrefetchScalarGridSpec(
            num_scalar_prefetch=1,
            grid=(nb,),
            in_specs=[
                pl.BlockSpec((BT, I),
                             lambda b, tb: (tb[1, b], 0)),
                pl.BlockSpec((1, H, I),
                             lambda b, tb: (tb[2, b], 0, 0)),
            ],
            out_specs=pl.BlockSpec((BT, H),
                                   lambda b, tb: (tb[1, b], 0)),
        ),
        out_shape=jax.ShapeDtypeStruct((P, H), jnp.float32),
        compiler_params=pltpu.CompilerParams(
            dimension_semantics=("arbitrary",),
        ),
    )
    out_d = g2(tab, hmat, w2_weight)

    tchunk = T // NW
    w3 = topk_weights.reshape(NW, (tchunk * K) // 16, 16)
    combine = pl.kernel(
        functools.partial(_sc_combine_body, K, H, tchunk),
        mesh=mesh,
        out_type=[jax.ShapeDtypeStruct((T, H), jnp.float32)],
        scratch_types=[
            pltpu.VMEM(((tchunk * K) // 16, 16), jnp.int32),
            pltpu.VMEM(((tchunk * K) // 16, 16), jnp.float32),
            pltpu.VMEM((2, 16, H), jnp.float32),
            pltpu.VMEM((16 // K, H), jnp.float32),
            pltpu.VMEM((2, 16), jnp.int32),
            pltpu.SemaphoreType.DMA,
        ],
    )
    (out,) = combine(pos3, w3, out_d)
    return out
